# speedup vs baseline: 157.9961x; 157.9961x over previous
import numpy as np
import jax
import jax.numpy as jnp
from jax.experimental import pallas as pl

H = 256
W = 256
STRIP = 32
NSTRIP = H // STRIP
PAD = 8
PH = H + 2 * PAD


def _spiral_coords(sr):
    coords = [(0, 0)]
    y = x = 0
    moves = [(0, 1), (1, 0), (0, -1), (-1, 0)]
    step = 1
    d = 0
    sz = 2 * sr + 1
    while len(coords) < sz * sz:
        for _ in range(2):
            dy, dx = moves[d % 4]
            for _ in range(step):
                y += dy
                x += dx
                if abs(y) <= sr and abs(x) <= sr and len(coords) < sz * sz:
                    coords.append((y, x))
            d += 1
        step += 1
    return coords


_SPIRAL = _spiral_coords(3)


def _border_sum(P):
    c5 = P[:, 0:256] + P[:, 1:257] + P[:, 2:258] + P[:, 3:259] + P[:, 4:260]
    c3 = P[:, 1:257] + P[:, 2:258] + P[:, 3:259]
    r5 = c5[0:34] + c5[1:35] + c5[2:36] + c5[3:37] + c5[4:38]
    r3 = c3[1:35] + c3[2:36] + c3[3:37]
    return r5 - r3


def _flow_kernel(fp_ref, gp_ref, out_ref):
    pid = pl.program_id(0)
    r0 = pid * STRIP
    fpad = fp_ref[pl.ds(r0, 48), :]
    gpad = gp_ref[pl.ds(r0, 48), :]

    def smooth_q(x):
        v = (x[0:46, :] + 2.0 * x[1:47, :] + x[2:48, :]) * 0.25
        h = (v[:, 0:270] + 2.0 * v[:, 1:271] + v[:, 2:272]) * 0.25
        return jnp.clip(jnp.round(h * 255.0), 0.0, 255.0)

    zrow = jnp.zeros((1, 270), jnp.float32)
    zcol = jnp.zeros((48, 1), jnp.float32)

    def embed(q):
        q = jnp.concatenate([zrow, q, zrow], axis=0)
        return jnp.concatenate([zcol, q, zcol], axis=1)

    rows48 = jax.lax.broadcasted_iota(jnp.int32, (48, 272), 0) + (r0 - 8)
    cols48 = jax.lax.broadcasted_iota(jnp.int32, (48, 272), 1) - 8
    inimg = (rows48 >= 0) & (rows48 < H) & (cols48 >= 0) & (cols48 < W)
    fz = jnp.where(inimg, embed(smooth_q(fpad)), 0.0)
    gz = jnp.where(inimg, embed(smooth_q(gpad)), 0.0)
    fs = fz / 255.0
    gs = gz / 255.0

    Xc = fs[5:43, :]
    Xd = fs[6:44, :]
    Xu = fs[4:42, :]
    rowsQ = jax.lax.broadcasted_iota(jnp.int32, (38, 272), 0) + (r0 - 3)
    colsQ = jax.lax.broadcasted_iota(jnp.int32, (38, 272), 1) - 8
    dfy = (jnp.where(rowsQ == H - 1, Xc, Xd) - jnp.where(rowsQ == 0, Xc, Xu)) * 0.5
    Xr = jnp.concatenate([Xc[:, 1:], Xc[:, 271:272]], axis=1)
    Xl = jnp.concatenate([Xc[:, 0:1], Xc[:, 0:271]], axis=1)
    dfx = (jnp.where(colsQ == W - 1, Xc, Xr) - jnp.where(colsQ == 0, Xc, Xl)) * 0.5
    qmask = (rowsQ >= 0) & (rowsQ < H) & (colsQ >= 0) & (colsQ < W)
    dfy = jnp.where(qmask, dfy, 0.0)
    dfx = jnp.where(qmask, dfx, 0.0)
    DX = dfx[:, 6:266]
    DY = dfy[:, 6:266]

    F2 = fz[5:43, 6:266]
    FS2 = fs[5:43, 6:266]

    bc = vy = vx = Px = Py = None
    for (jy, jx) in _SPIRAL:
        Gq = gz[5 + jy:43 + jy, 6 + jx:266 + jx]
        E = jnp.abs(Gq - F2)
        c5 = E[:, 0:256] + E[:, 1:257] + E[:, 2:258] + E[:, 3:259] + E[:, 4:260]
        cost = c5[0:34] + c5[1:35] + c5[2:36] + c5[3:37] + c5[4:38]
        Gv = gs[5 + jy:43 + jy, 6 + jx:266 + jx]
        px = _border_sum(Gv * DX)
        py = _border_sum(Gv * DY)
        if bc is None:
            bc = cost
            vy = jnp.zeros_like(cost)
            vx = jnp.zeros_like(cost)
            Px = px
            Py = py
        else:
            m = cost < bc
            bc = jnp.where(m, cost, bc)
            vy = jnp.where(m, float(-jy), vy)
            vx = jnp.where(m, float(-jx), vx)
            Px = jnp.where(m, px, Px)
            Py = jnp.where(m, py, Py)

    A = _border_sum(DX * DX)
    Bv = _border_sum(DX * DY)
    Dv = _border_sum(DY * DY)
    Fx = _border_sum(FS2 * DX)
    Fy = _border_sum(FS2 * DY)
    p = Px - Fx
    q = Py - Fy
    det = A * Dv - Bv * Bv
    bad = det <= 1e-7
    sd = jnp.where(bad, 1.0, det)
    u = (Dv * p - Bv * q) / sd
    v = (A * q - Bv * p) / sd
    u = jnp.where(bad | (jnp.abs(u) >= 1.0), 0.0, u)
    v = jnp.where(bad | (jnp.abs(v) >= 1.0), 0.0, v)
    fl0 = vy + v
    fl1 = vx + u

    rows32 = jax.lax.broadcasted_iota(jnp.int32, (32, 256), 0) + r0

    def med3(a, b, c):
        return jnp.maximum(jnp.minimum(a, b), jnp.minimum(jnp.maximum(a, b), c))

    def sort3(a, b, c):
        lo = jnp.minimum(a, b)
        hi = jnp.maximum(a, b)
        mx = jnp.maximum(hi, c)
        m2 = jnp.minimum(hi, c)
        return jnp.minimum(lo, m2), jnp.maximum(lo, m2), mx

    def median9(ch):
        cur = ch[1:33]
        up = jnp.where(rows32 == 0, cur, ch[0:32])
        dn = jnp.where(rows32 == H - 1, cur, ch[2:34])

        def lr(t):
            L = jnp.concatenate([t[:, 0:1], t[:, 0:255]], axis=1)
            R = jnp.concatenate([t[:, 1:256], t[:, 255:256]], axis=1)
            return L, t, R

        mn0, md0, mx0 = sort3(*lr(up))
        mn1, md1, mx1 = sort3(*lr(cur))
        mn2, md2, mx2 = sort3(*lr(dn))
        return med3(jnp.maximum(jnp.maximum(mn0, mn1), mn2),
                    med3(md0, md1, md2),
                    jnp.minimum(jnp.minimum(mx0, mx1), mx2))

    out_ref[0, 0, :, :] = median9(fl0)
    out_ref[0, 1, :, :] = median9(fl1)


def _run(fp, gp, interpret=False):
    return pl.pallas_call(
        _flow_kernel,
        grid=(NSTRIP,),
        in_specs=[pl.BlockSpec((PH, PH), lambda i: (0, 0)),
                  pl.BlockSpec((PH, PH), lambda i: (0, 0))],
        out_specs=pl.BlockSpec((1, 2, STRIP, W), lambda i: (0, 0, i, 0)),
        out_shape=jax.ShapeDtypeStruct((1, 2, H, W), jnp.float32),
        interpret=interpret,
    )(fp, gp)


def kernel(f, g):
    fp = jnp.pad(f[0, 0], PAD, mode='edge')
    gp = jnp.pad(g[0, 0], PAD, mode='edge')
    return _run(fp, gp)
